# baseline (speedup 1.0000x reference)
import functools

import jax
import jax.numpy as jnp
from jax import lax
from jax.experimental import pallas as pl
from jax.experimental.pallas import tpu as pltpu
from jax.experimental.pallas import tpu_sc as plsc

V, E, NCLS, B, S = 1000, 64, 10, 4096, 200
NC, NS, L = 2, 16, 16
NW = NC * NS
BPW = B // NW
NFULL = S // L
NREM = S - NFULL * L
NG = BPW // L
EW = E // 2

_PERM = ([2 * i for i in range(16)] + [2 * i + 1 for i in range(16)]
         + [32 + 2 * i for i in range(16)] + [33 + 2 * i for i in range(16)])

_mesh = plsc.VectorSubcoreMesh(core_axis_name="c", subcore_axis_name="s")


def _bcast_lane(vec, j):
    return lax.gather(
        vec,
        jnp.full((L, 1), j, jnp.int32),
        lax.GatherDimensionNumbers(
            offset_dims=(), collapsed_slice_dims=(0,), start_index_map=(0,)),
        (1,),
        mode=lax.GatherScatterMode.PROMISE_IN_BOUNDS,
    )


@functools.partial(
    pl.kernel,
    out_type=jax.ShapeDtypeStruct((B * NCLS,), jnp.float32),
    mesh=_mesh,
    scratch_types=[
        pltpu.VMEM((BPW * S + L - NREM,), jnp.int32),
        pltpu.VMEM((V * EW,), jnp.int32),
        pltpu.VMEM((E * L,), jnp.float32),
        pltpu.VMEM((L,), jnp.float32),
        pltpu.VMEM((L * E,), jnp.float32),
        pltpu.VMEM((BPW * NCLS,), jnp.float32),
    ],
    compiler_params=pltpu.CompilerParams(needs_layout_passes=False),
)
def _xswem_sc(idx_hbm, tbl_hbm, w_hbm, b_hbm, out_hbm,
              idx_v, tbl_v, w_v, b_v, pool_v, out_v):
    wid = lax.axis_index("s") * NC + lax.axis_index("c")
    base = wid * BPW
    pltpu.sync_copy(tbl_hbm, tbl_v)
    pltpu.sync_copy(w_hbm, w_v)
    pltpu.sync_copy(b_hbm, b_v)
    pltpu.sync_copy(idx_hbm.at[pl.ds(base * S, BPW * S)],
                    idx_v.at[pl.ds(0, BPW * S)])
    lanes = lax.iota(jnp.int32, L)
    ninf = jnp.full((2 * L,), -jnp.inf, jnp.bfloat16)

    def gather_max(idxv, j, a01):
        a0, a1 = a01
        addr = _bcast_lane(idxv, j) * EW + lanes
        w0 = plsc.bitcast(plsc.load_gather(tbl_v, [addr]), jnp.bfloat16)
        w1 = plsc.bitcast(plsc.load_gather(tbl_v, [addr + L]), jnp.bfloat16)
        return jnp.maximum(a0, w0), jnp.maximum(a1, w1)

    def row_body(rr, g):
        row = g * L + rr

        def chunk_body(c, accs):
            idxv = idx_v[pl.ds(row * S + c * L, L)]
            e01, o01 = accs[:2], accs[2:]
            for j in range(L):
                if j % 2 == 0:
                    e01 = gather_max(idxv, j, e01)
                else:
                    o01 = gather_max(idxv, j, o01)
            return e01 + o01

        accs = lax.fori_loop(0, NFULL, chunk_body, (ninf,) * 4)
        idxv = idx_v[pl.ds(row * S + NFULL * L, L)]
        e01, o01 = accs[:2], accs[2:]
        for j in range(NREM):
            if j % 2 == 0:
                e01 = gather_max(idxv, j, e01)
            else:
                o01 = gather_max(idxv, j, o01)
        lo = jnp.maximum(e01[0], o01[0])
        hi = jnp.maximum(e01[1], o01[1])
        l0, l1 = plsc.unpack(lo, format=plsc.PackFormat.INTERLEAVED)
        h0, h1 = plsc.unpack(hi, format=plsc.PackFormat.INTERLEAVED)
        pool_v[pl.ds(rr * E, L)] = l0
        pool_v[pl.ds(rr * E + L, L)] = l1
        pool_v[pl.ds(rr * E + 2 * L, L)] = h0
        pool_v[pl.ds(rr * E + 3 * L, L)] = h1
        return g

    bvec = b_v[...]

    def group_body(g, _):
        lax.fori_loop(0, L, row_body, g)
        logits = [_bcast_lane(bvec, c) for c in range(NCLS)]

        def dense_body(e, logits):
            col = plsc.load_gather(pool_v, [lanes * E + e])
            wvec = w_v[pl.ds(e * L, L)]
            return tuple(logits[c] + col * _bcast_lane(wvec, c)
                         for c in range(NCLS))

        logits = lax.fori_loop(0, E, dense_body, tuple(logits))
        m = logits[0]
        for c in range(1, NCLS):
            m = jnp.maximum(m, logits[c])
        exps = [jnp.exp(lc - m) for lc in logits]
        s = exps[0]
        for c in range(1, NCLS):
            s = s + exps[c]
        rows = g * L + lanes
        for c in range(NCLS):
            plsc.store_scatter(out_v, [rows * NCLS + c], exps[c] / s)
        return 0

    lax.fori_loop(0, NG, group_body, 0)
    pltpu.sync_copy(out_v, out_hbm.at[pl.ds(base * NCLS, BPW * NCLS)])


def kernel(indices, table, W, b):
    tbl_p = lax.bitcast_convert_type(
        table.astype(jnp.bfloat16).reshape(V, EW, 2), jnp.int32).reshape(-1)
    w_p = jnp.pad(W[jnp.array(_PERM), :],
                  ((0, 0), (0, L - NCLS))).reshape(-1)
    b_p = jnp.pad(b, (0, L - NCLS))
    probs = _xswem_sc(indices.reshape(-1), tbl_p, w_p, b_p)
    return probs.reshape(B, NCLS)


# speedup vs baseline: 1.0004x; 1.0004x over previous
import functools

import jax
import jax.numpy as jnp
from jax import lax
from jax.experimental import pallas as pl
from jax.experimental.pallas import tpu as pltpu
from jax.experimental.pallas import tpu_sc as plsc

V, E, NCLS, B, S = 1000, 64, 10, 4096, 200
NC, NS, L = 2, 16, 16
NW = NC * NS
BPW = B // NW
NFULL = S // L
NREM = S - NFULL * L
EW = E // 2
CPAD = 128

_mesh = plsc.VectorSubcoreMesh(core_axis_name="c", subcore_axis_name="s")


def _bcast_lane(vec, j):
    return lax.gather(
        vec,
        jnp.full((L, 1), j, jnp.int32),
        lax.GatherDimensionNumbers(
            offset_dims=(), collapsed_slice_dims=(0,), start_index_map=(0,)),
        (1,),
        mode=lax.GatherScatterMode.PROMISE_IN_BOUNDS,
    )


@functools.partial(
    pl.kernel,
    out_type=jax.ShapeDtypeStruct((B * EW,), jnp.int32),
    mesh=_mesh,
    scratch_types=[
        pltpu.VMEM((BPW * S + L - NREM,), jnp.int32),
        pltpu.VMEM((V * EW,), jnp.int32),
        pltpu.VMEM((BPW * EW,), jnp.int32),
    ],
    compiler_params=pltpu.CompilerParams(needs_layout_passes=False),
)
def _pool_sc(idx_hbm, tbl_hbm, out_hbm, idx_v, tbl_v, out_v):
    wid = lax.axis_index("s") * NC + lax.axis_index("c")
    base = wid * BPW
    pltpu.sync_copy(tbl_hbm, tbl_v)
    pltpu.sync_copy(idx_hbm.at[pl.ds(base * S, BPW * S)],
                    idx_v.at[pl.ds(0, BPW * S)])
    lanes = lax.iota(jnp.int32, L)
    ninf = jnp.full((2 * L,), -jnp.inf, jnp.bfloat16)

    def gather_max(addr, acc):
        row = plsc.bitcast(plsc.load_gather(tbl_v, [addr]), jnp.bfloat16)
        return jnp.maximum(acc, row)

    def row_body(row, _):
        def chunk_body(c, accs):
            idxv = idx_v[pl.ds(row * S + c * L, L)]
            a, b2 = accs
            for j in range(L):
                addr = _bcast_lane(idxv, j) * EW + lanes
                a = gather_max(addr, a)
                b2 = gather_max(addr + L, b2)
            return (a, b2)

        accs = lax.fori_loop(0, NFULL, chunk_body, (ninf, ninf))
        idxv = idx_v[pl.ds(row * S + NFULL * L, L)]
        a, b2 = accs
        for j in range(NREM):
            addr = _bcast_lane(idxv, j) * EW + lanes
            a = gather_max(addr, a)
            b2 = gather_max(addr + L, b2)
        out_v[pl.ds(row * EW, L)] = plsc.bitcast(a, jnp.int32)
        out_v[pl.ds(row * EW + L, L)] = plsc.bitcast(b2, jnp.int32)
        return 0

    lax.fori_loop(0, BPW, row_body, 0)
    pltpu.sync_copy(out_v, out_hbm.at[pl.ds(base * EW, BPW * EW)])


BLK = 512


def _dense_tc(x_ref, w_ref, b_ref, o_ref):
    logits = jnp.dot(x_ref[...], w_ref[...],
                     preferred_element_type=jnp.float32) + b_ref[...]
    m = jnp.max(logits, axis=1, keepdims=True)
    e = jnp.exp(logits - m)
    o_ref[...] = e / jnp.sum(e, axis=1, keepdims=True)


_dense_call = pl.pallas_call(
    _dense_tc,
    grid=(B // BLK,),
    in_specs=[
        pl.BlockSpec((BLK, E), lambda i: (i, 0)),
        pl.BlockSpec((E, CPAD), lambda i: (0, 0)),
        pl.BlockSpec((1, CPAD), lambda i: (0, 0)),
    ],
    out_specs=pl.BlockSpec((BLK, CPAD), lambda i: (i, 0)),
    out_shape=jax.ShapeDtypeStruct((B, CPAD), jnp.float32),
)


def kernel(indices, table, W, b):
    tbl_p = lax.bitcast_convert_type(
        table.astype(jnp.bfloat16).reshape(V, EW, 2), jnp.int32).reshape(-1)
    pooled = _pool_sc(indices.reshape(-1), tbl_p)
    x = lax.bitcast_convert_type(
        pooled.reshape(B, EW), jnp.bfloat16).reshape(B, E)
    w_p = jnp.pad(W.astype(jnp.bfloat16), ((0, 0), (0, CPAD - NCLS)))
    b_p = jnp.concatenate(
        [b, jnp.full((CPAD - NCLS,), -1e30, jnp.float32)]).reshape(1, CPAD)
    return _dense_call(x, w_p, b_p)[:, :NCLS]
